# Initial kernel scaffold; baseline (speedup 1.0000x reference)
import jax
import jax.numpy as jnp
from jax.experimental import pallas as pl


def kernel(feature, label, embedding_weight):
    raise NotImplementedError("write your pallas kernel here")


# speedup vs baseline: 2.1733x; 2.1733x over previous
import functools

import jax
import jax.numpy as jnp
from jax import lax
from jax.experimental import pallas as pl
from jax.experimental.pallas import tpu as pltpu
from jax.experimental.pallas import tpu_sc as plsc

N = 16384
NUM_CORES = 2
NUM_SUBCORES = 16
NUM_WORKERS = NUM_CORES * NUM_SUBCORES
SAMPLES_PER_TILE = N // NUM_WORKERS
FLAT_PER_TILE = SAMPLES_PER_TILE * 2
VECS_PER_TILE = FLAT_PER_TILE // 16
TBL_PAD = 32


def _center_loss_body(feature_hbm, label_hbm, table_hbm, out_hbm,
                      f_vmem, lab_vmem, tbl_vmem, acc_vmem,
                      shared, core_vmem, out_vmem):
    cid = lax.axis_index("c")
    sid = lax.axis_index("s")
    wid = cid * NUM_SUBCORES + sid

    pltpu.sync_copy(feature_hbm.at[pl.ds(wid * FLAT_PER_TILE, FLAT_PER_TILE)],
                    f_vmem)
    pltpu.sync_copy(label_hbm.at[pl.ds(wid * SAMPLES_PER_TILE,
                                       SAMPLES_PER_TILE)], lab_vmem)
    pltpu.sync_copy(table_hbm, tbl_vmem)

    lane = lax.iota(jnp.int32, 16)

    def step(j, acc):
        base = j * 16
        f = f_vmem[pl.ds(base, 16)]
        flat = base + lane
        lab = plsc.load_gather(lab_vmem, [lax.shift_right_logical(flat, 1)])
        cval = plsc.load_gather(tbl_vmem, [lab * 2 + (flat & 1)])
        d = f - cval
        return acc + d * d

    acc = lax.fori_loop(0, VECS_PER_TILE, step,
                        jnp.zeros((16,), jnp.float32))

    acc_vmem[...] = acc
    pltpu.sync_copy(acc_vmem, shared.at[pl.ds(wid * 16, 16)])
    plsc.subcore_barrier()

    @pl.when(sid == 0)
    def _():
        pltpu.sync_copy(shared.at[pl.ds(cid * NUM_SUBCORES * 16,
                                        NUM_SUBCORES * 16)], core_vmem)
        tot = jnp.zeros((16,), jnp.float32)
        for i in range(NUM_SUBCORES):
            tot = tot + core_vmem[pl.ds(i * 16, 16)]
        out_vmem[...] = tot
        pltpu.sync_copy(out_vmem, out_hbm.at[cid])


@jax.jit
def _center_loss(feature_flat, label_i32, table_pad):
    mesh = plsc.VectorSubcoreMesh(core_axis_name="c", subcore_axis_name="s")
    run = functools.partial(
        pl.kernel,
        mesh=mesh,
        compiler_params=pltpu.CompilerParams(needs_layout_passes=False),
        out_type=jax.ShapeDtypeStruct((NUM_CORES, 16), jnp.float32),
        scratch_types=[
            pltpu.VMEM((FLAT_PER_TILE,), jnp.float32),
            pltpu.VMEM((SAMPLES_PER_TILE,), jnp.int32),
            pltpu.VMEM((TBL_PAD,), jnp.float32),
            pltpu.VMEM((16,), jnp.float32),
            pltpu.VMEM_SHARED((NUM_WORKERS * 16,), jnp.float32),
            pltpu.VMEM((NUM_SUBCORES * 16,), jnp.float32),
            pltpu.VMEM((16,), jnp.float32),
        ],
    )(_center_loss_body)
    out = run(feature_flat, label_i32, table_pad)
    return jnp.sum(out) * (1.0 / N)


def kernel(feature, label, embedding_weight):
    feature_flat = feature.reshape(-1)
    label_i32 = label.astype(jnp.int32)
    table_pad = jnp.zeros((TBL_PAD,), jnp.float32).at[:20].set(
        embedding_weight.reshape(-1))
    return _center_loss(feature_flat, label_i32, table_pad)


# speedup vs baseline: 2.5411x; 1.1693x over previous
import functools

import jax
import jax.numpy as jnp
from jax import lax
from jax.experimental import pallas as pl
from jax.experimental.pallas import tpu as pltpu
from jax.experimental.pallas import tpu_sc as plsc

N = 16384
NUM_SUBCORES = 16
SAMPLES_PER_TILE = N // NUM_SUBCORES
FLAT_PER_TILE = SAMPLES_PER_TILE * 2
UNROLL = 4
VECS_PER_TILE = FLAT_PER_TILE // 16
STEPS = VECS_PER_TILE // UNROLL


def _center_loss_body(feature_hbm, label_hbm, table_hbm, out_hbm,
                      f_vmem, lab_vmem, tbl_vmem, acc_vmem,
                      shared, core_vmem, out_vmem, sem):
    sid = lax.axis_index("s")

    cp_f = pltpu.make_async_copy(
        feature_hbm.at[pl.ds(sid * FLAT_PER_TILE, FLAT_PER_TILE)],
        f_vmem, sem)
    cp_l = pltpu.make_async_copy(
        label_hbm.at[pl.ds(sid * SAMPLES_PER_TILE, SAMPLES_PER_TILE)],
        lab_vmem, sem)
    cp_t = pltpu.make_async_copy(table_hbm, tbl_vmem, sem)
    cp_f.start()
    cp_l.start()
    cp_t.start()
    cp_f.wait()
    cp_l.wait()
    cp_t.wait()

    lane = lax.iota(jnp.int32, 16)

    def step(j, accs):
        base = j * (16 * UNROLL)
        out = []
        for u in range(UNROLL):
            b = base + u * 16
            f = f_vmem[pl.ds(b, 16)]
            flat = b + lane
            lab = plsc.load_gather(lab_vmem,
                                   [lax.shift_right_logical(flat, 1)])
            cval = plsc.load_gather(tbl_vmem, [lab * 2 + (flat & 1)])
            d = f - cval
            out.append(accs[u] + d * d)
        return tuple(out)

    accs = lax.fori_loop(0, STEPS, step,
                         tuple(jnp.zeros((16,), jnp.float32)
                               for _ in range(UNROLL)))
    acc = accs[0] + accs[1] + accs[2] + accs[3]

    acc_vmem[...] = acc
    pltpu.sync_copy(acc_vmem, shared.at[pl.ds(sid * 16, 16)])
    plsc.subcore_barrier()

    @pl.when(sid == 0)
    def _():
        pltpu.sync_copy(shared, core_vmem)
        tot = jnp.zeros((16,), jnp.float32)
        for i in range(NUM_SUBCORES):
            tot = tot + core_vmem[pl.ds(i * 16, 16)]
        s = jnp.sum(tot) * (1.0 / N)
        out_vmem[...] = jnp.full((16,), s, jnp.float32)
        pltpu.sync_copy(out_vmem.at[pl.ds(0, 8)], out_hbm)


@jax.jit
def _center_loss(feature_flat, label_i32, table_flat):
    mesh = plsc.VectorSubcoreMesh(core_axis_name="c", subcore_axis_name="s",
                                  num_cores=1)
    run = functools.partial(
        pl.kernel,
        mesh=mesh,
        compiler_params=pltpu.CompilerParams(needs_layout_passes=False),
        out_type=jax.ShapeDtypeStruct((8,), jnp.float32),
        scratch_types=[
            pltpu.VMEM((FLAT_PER_TILE,), jnp.float32),
            pltpu.VMEM((SAMPLES_PER_TILE,), jnp.int32),
            pltpu.VMEM((20,), jnp.float32),
            pltpu.VMEM((16,), jnp.float32),
            pltpu.VMEM_SHARED((NUM_SUBCORES * 16,), jnp.float32),
            pltpu.VMEM((NUM_SUBCORES * 16,), jnp.float32),
            pltpu.VMEM((16,), jnp.float32),
            pltpu.SemaphoreType.DMA,
        ],
    )(_center_loss_body)
    out = run(feature_flat, label_i32, table_flat)
    return out[0]


def kernel(feature, label, embedding_weight):
    feature_flat = feature.reshape(-1)
    label_i32 = label.astype(jnp.int32)
    table_flat = embedding_weight.reshape(-1)
    return _center_loss(feature_flat, label_i32, table_flat)
